# Initial kernel scaffold; baseline (speedup 1.0000x reference)
import functools

import jax
import jax.numpy as jnp
from jax import lax
from jax.experimental import pallas as pl
from jax.experimental.pallas import tpu as pltpu
from jax.experimental.pallas import tpu_sc as plsc

N = 10000
E = 320000
D = 128
DH = D // 2

NC = 2
NS = 16

NPAD = 10240
RPT = NPAD // NS
CHUNK = 128
EPAD = 323584
NCH_DEG = EPAD // (NC * NS) // CHUNK
NCH_ACC = EPAD // NS // CHUNK

_MESH = plsc.VectorSubcoreMesh(core_axis_name="c", subcore_axis_name="s")


def _deg_body(dst_hbm, deg2_hbm, idx_v, ones_v, zero_v, deg_sh):
    c = lax.axis_index("c")
    s = lax.axis_index("s")
    for i in range(CHUNK // 16):
        ones_v[pl.ds(i * 16, 16)] = jnp.ones((16,), jnp.float32)
    for i in range(RPT // 16):
        zero_v[pl.ds(i * 16, 16)] = jnp.zeros((16,), jnp.float32)
    pltpu.sync_copy(zero_v, deg_sh.at[pl.ds(s * RPT, RPT)])
    plsc.subcore_barrier()

    wid = c * NS + s

    @pl.loop(0, NCH_DEG)
    def _(j):
        pltpu.sync_copy(dst_hbm.at[wid, j], idx_v)
        pltpu.sync_copy(ones_v, deg_sh.at[idx_v], add=True)

    plsc.subcore_barrier()
    pltpu.sync_copy(deg_sh.at[pl.ds(s * RPT, RPT)],
                    deg2_hbm.at[c, pl.ds(s * RPT, RPT)])


_deg_call = pl.kernel(
    _deg_body,
    out_type=jax.ShapeDtypeStruct((NC, NPAD), jnp.float32),
    mesh=_MESH,
    scratch_types=[
        pltpu.VMEM((CHUNK,), jnp.int32),
        pltpu.VMEM((CHUNK,), jnp.float32),
        pltpu.VMEM((RPT,), jnp.float32),
        pltpu.VMEM_SHARED((NPAD,), jnp.float32),
    ],
)


def _edge_body(srcx_hbm, dst_hbm, y2f_hbm, acc2_hbm,
               sidx_v, didx_v, rows_v, acc_sh):
    c = lax.axis_index("c")
    s = lax.axis_index("s")
    pltpu.sync_copy(y2f_hbm.at[pl.ds(c * NPAD + s * RPT, RPT)],
                    acc_sh.at[pl.ds(s * RPT, RPT)])
    plsc.subcore_barrier()

    @pl.loop(0, NCH_ACC)
    def _(j):
        pltpu.sync_copy(srcx_hbm.at[c, s, j], sidx_v)
        pltpu.sync_copy(dst_hbm.at[s, j], didx_v)
        pltpu.sync_copy(y2f_hbm.at[sidx_v], rows_v)
        pltpu.sync_copy(rows_v, acc_sh.at[didx_v], add=True)

    plsc.subcore_barrier()
    pltpu.sync_copy(acc_sh.at[pl.ds(s * RPT, RPT)],
                    acc2_hbm.at[c, pl.ds(s * RPT, RPT)])


_edge_call = pl.kernel(
    _edge_body,
    out_type=jax.ShapeDtypeStruct((NC, NPAD, DH), jnp.float32),
    mesh=_MESH,
    scratch_types=[
        pltpu.VMEM((CHUNK,), jnp.int32),
        pltpu.VMEM((CHUNK,), jnp.int32),
        pltpu.VMEM((CHUNK, DH), jnp.float32),
        pltpu.VMEM_SHARED((NPAD, DH), jnp.float32),
    ],
)


def _mm_body(x_ref, w_ref, deg2_ref, y2_ref):
    xw = jnp.dot(x_ref[...], w_ref[...], preferred_element_type=jnp.float32)
    deg = deg2_ref[0] + deg2_ref[1] + 1.0
    dis = lax.rsqrt(deg)
    y = xw * dis[:, None]
    y2_ref[0] = y[:, :DH]
    y2_ref[1] = y[:, DH:]


_MM_BLK = 1280


def _mm_call(x_pad, w, deg2):
    grid = NPAD // _MM_BLK
    return pl.pallas_call(
        _mm_body,
        grid=(grid,),
        in_specs=[
            pl.BlockSpec((_MM_BLK, D), lambda i: (i, 0)),
            pl.BlockSpec((D, D), lambda i: (0, 0)),
            pl.BlockSpec((NC, _MM_BLK), lambda i: (0, i)),
        ],
        out_specs=pl.BlockSpec((NC, _MM_BLK, DH), lambda i: (0, i, 0)),
        out_shape=jax.ShapeDtypeStruct((NC, NPAD, DH), jnp.float32),
    )(x_pad, w, deg2)


def _post_body(acc2_ref, deg2_ref, b_ref, gamma_ref, beta_ref, out_ref):
    deg = deg2_ref[0, :N] + deg2_ref[1, :N] + 1.0
    dis = lax.rsqrt(deg)[:, None]
    for c in range(NC):
        pre = acc2_ref[c, :N, :] * dis + b_ref[0, c * DH:(c + 1) * DH]
        pre = jnp.where(pre >= 0, pre, 0.01 * pre)
        mean = jnp.mean(pre, axis=0, keepdims=True)
        cent = pre - mean
        var = jnp.mean(cent * cent, axis=0, keepdims=True)
        scale = lax.rsqrt(var + 1e-5) * gamma_ref[0, c * DH:(c + 1) * DH]
        out_ref[:, c * DH:(c + 1) * DH] = cent * scale + beta_ref[0, c * DH:(c + 1) * DH]


def _post_call(acc2, deg2, b, gamma, beta):
    return pl.pallas_call(
        _post_body,
        out_shape=jax.ShapeDtypeStruct((N, D), jnp.float32),
    )(acc2, deg2, b.reshape(1, D), gamma.reshape(1, D), beta.reshape(1, D))


def kernel(x, edge_index, W, b, gamma, beta):
    ei = edge_index.astype(jnp.int32)
    src, dst = ei[0], ei[1]
    pad_ids = N + (jnp.arange(EPAD - E, dtype=jnp.int32) % (NPAD - N))
    src_p = jnp.concatenate([src, pad_ids])
    dst_p = jnp.concatenate([dst, pad_ids])

    dst32 = dst_p.reshape(NC * NS, NCH_DEG, CHUNK)
    dst16 = dst_p.reshape(NS, NCH_ACC, CHUNK)
    srcx = jnp.stack([src_p, src_p + NPAD]).reshape(NC, NS, NCH_ACC, CHUNK)

    x_pad = jnp.pad(x, ((0, NPAD - N), (0, 0)))

    deg2 = _deg_call(dst32)
    y2 = _mm_call(x_pad, W, deg2)
    acc2 = _edge_call(srcx, dst16, y2.reshape(NC * NPAD, DH))
    return _post_call(acc2, deg2, b, gamma, beta)


# speedup vs baseline: 22.6601x; 22.6601x over previous
import jax
import jax.numpy as jnp
from jax import lax
from jax.experimental import pallas as pl
from jax.experimental.pallas import tpu as pltpu
from jax.experimental.pallas import tpu_sc as plsc

N = 10000
E = 320000
D = 128

NC = 2
NS = 16

NPAD = 10240
RPT = NPAD // NS
CHUNK = 128
EPAD = 323584
NCH = EPAD // (NC * NS) // CHUNK

_MESH = plsc.VectorSubcoreMesh(core_axis_name="c", subcore_axis_name="s")


def _deg_body(dst_hbm, deg2_hbm, idx_v, ones_v, zero_v, deg_sh):
    c = lax.axis_index("c")
    s = lax.axis_index("s")
    for i in range(CHUNK // 16):
        ones_v[pl.ds(i * 16, 16)] = jnp.ones((16,), jnp.float32)
    for i in range(RPT // 16):
        zero_v[pl.ds(i * 16, 16)] = jnp.zeros((16,), jnp.float32)
    pltpu.sync_copy(zero_v, deg_sh.at[pl.ds(s * RPT, RPT)])
    plsc.subcore_barrier()

    @pl.loop(0, NCH)
    def _(j):
        pltpu.sync_copy(dst_hbm.at[c, s, j], idx_v)
        pltpu.sync_copy(ones_v, deg_sh.at[idx_v], add=True)

    plsc.subcore_barrier()
    pltpu.sync_copy(deg_sh.at[pl.ds(s * RPT, RPT)],
                    deg2_hbm.at[c, pl.ds(s * RPT, RPT)])


_deg_call = pl.kernel(
    _deg_body,
    out_type=jax.ShapeDtypeStruct((NC, NPAD), jnp.float32),
    mesh=_MESH,
    scratch_types=[
        pltpu.VMEM((CHUNK,), jnp.int32),
        pltpu.VMEM((CHUNK,), jnp.float32),
        pltpu.VMEM((RPT,), jnp.float32),
        pltpu.VMEM_SHARED((NPAD,), jnp.float32),
    ],
)


def _edge_body(src_hbm, dst_hbm, y_hbm, z_hbm, acc2_hbm,
               sidx_v, didx_v, rows_v, acc_sh):
    c = lax.axis_index("c")
    s = lax.axis_index("s")

    @pl.when(c == 0)
    def _():
        pltpu.sync_copy(y_hbm.at[pl.ds(s * RPT, RPT)],
                        acc_sh.at[pl.ds(s * RPT, RPT)])

    @pl.when(c == 1)
    def _():
        pltpu.sync_copy(z_hbm, acc_sh.at[pl.ds(s * RPT, RPT)])

    plsc.subcore_barrier()

    @pl.loop(0, NCH)
    def _(j):
        pltpu.sync_copy(src_hbm.at[c, s, j], sidx_v)
        pltpu.sync_copy(dst_hbm.at[c, s, j], didx_v)
        pltpu.sync_copy(y_hbm.at[sidx_v], rows_v)
        pltpu.sync_copy(rows_v, acc_sh.at[didx_v], add=True)

    plsc.subcore_barrier()
    pltpu.sync_copy(acc_sh.at[pl.ds(s * RPT, RPT)],
                    acc2_hbm.at[c, pl.ds(s * RPT, RPT)])


_edge_call = pl.kernel(
    _edge_body,
    out_type=jax.ShapeDtypeStruct((NC, NPAD, D), jnp.float32),
    mesh=_MESH,
    scratch_types=[
        pltpu.VMEM((CHUNK,), jnp.int32),
        pltpu.VMEM((CHUNK,), jnp.int32),
        pltpu.VMEM((CHUNK, D), jnp.float32),
        pltpu.VMEM_SHARED((NPAD, D), jnp.float32),
    ],
)


def _mm_body(x_ref, w_ref, deg2_ref, y_ref):
    xw = jnp.dot(x_ref[...], w_ref[...], preferred_element_type=jnp.float32)
    deg = deg2_ref[0] + deg2_ref[1] + 1.0
    dis = lax.rsqrt(deg)
    y_ref[...] = xw * dis[:, None]


_MM_BLK = 1280


def _mm_call(x_pad, w, deg2):
    grid = NPAD // _MM_BLK
    return pl.pallas_call(
        _mm_body,
        grid=(grid,),
        in_specs=[
            pl.BlockSpec((_MM_BLK, D), lambda i: (i, 0)),
            pl.BlockSpec((D, D), lambda i: (0, 0)),
            pl.BlockSpec((NC, _MM_BLK), lambda i: (0, i)),
        ],
        out_specs=pl.BlockSpec((_MM_BLK, D), lambda i: (i, 0)),
        out_shape=jax.ShapeDtypeStruct((NPAD, D), jnp.float32),
    )(x_pad, w, deg2)


def _post_body(acc2_ref, deg2_ref, b_ref, gamma_ref, beta_ref, out_ref):
    deg = deg2_ref[0, :N] + deg2_ref[1, :N] + 1.0
    dis = lax.rsqrt(deg)[:, None]
    acc = acc2_ref[0, :N, :] + acc2_ref[1, :N, :]
    pre = acc * dis + b_ref[0]
    pre = jnp.where(pre >= 0, pre, 0.01 * pre)
    mean = jnp.mean(pre, axis=0, keepdims=True)
    cent = pre - mean
    var = jnp.mean(cent * cent, axis=0, keepdims=True)
    out_ref[...] = cent * (lax.rsqrt(var + 1e-5) * gamma_ref[0]) + beta_ref[0]


def _post_call(acc2, deg2, b, gamma, beta):
    return pl.pallas_call(
        _post_body,
        out_shape=jax.ShapeDtypeStruct((N, D), jnp.float32),
    )(acc2, deg2, b.reshape(1, D), gamma.reshape(1, D), beta.reshape(1, D))


def kernel(x, edge_index, W, b, gamma, beta):
    ei = edge_index.astype(jnp.int32)
    src, dst = ei[0], ei[1]
    pad_ids = N + (jnp.arange(EPAD - E, dtype=jnp.int32) % (NPAD - N))
    src_p = jnp.concatenate([src, pad_ids]).reshape(NC, NS, NCH, CHUNK)
    dst_p = jnp.concatenate([dst, pad_ids]).reshape(NC, NS, NCH, CHUNK)

    x_pad = jnp.pad(x, ((0, NPAD - N), (0, 0)))
    zeros = jnp.zeros((RPT, D), jnp.float32)

    deg2 = _deg_call(dst_p)
    y = _mm_call(x_pad, W, deg2)
    acc2 = _edge_call(src_p, dst_p, y, zeros)
    return _post_call(acc2, deg2, b, gamma, beta)


# speedup vs baseline: 36.6164x; 1.6159x over previous
import jax
import jax.numpy as jnp
from jax import lax
from jax.experimental import pallas as pl
from jax.experimental.pallas import tpu as pltpu
from jax.experimental.pallas import tpu_sc as plsc

N = 10000
E = 320000
D = 128

NC = 2
NS = 16

NPAD = 10240
RPT = NPAD // NS
CHUNK = 64
EPAD = 327680
NCH = EPAD // (NC * NS) // CHUNK
NBUF = 4
DEGB = 8

_MESH = plsc.VectorSubcoreMesh(core_axis_name="c", subcore_axis_name="s")


def _deg_body(dst_hbm, deg2_hbm, idx_all, ones_v, zero_v, deg_sh, ssem):
    c = lax.axis_index("c")
    s = lax.axis_index("s")
    for i in range(CHUNK // 16):
        ones_v[pl.ds(i * 16, 16)] = jnp.ones((16,), jnp.float32)
    for i in range(RPT // 16):
        zero_v[pl.ds(i * 16, 16)] = jnp.zeros((16,), jnp.float32)
    pltpu.sync_copy(dst_hbm.at[c, s], idx_all)
    pltpu.sync_copy(zero_v, deg_sh.at[pl.ds(s * RPT, RPT)])
    plsc.subcore_barrier()

    @pl.loop(0, NCH // DEGB)
    def _(p):
        descs = [
            pltpu.async_copy(ones_v, deg_sh.at[idx_all.at[p * DEGB + b]],
                             ssem, add=True)
            for b in range(DEGB)
        ]
        for d in descs:
            d.wait()

    plsc.subcore_barrier()
    pltpu.sync_copy(deg_sh.at[pl.ds(s * RPT, RPT)],
                    deg2_hbm.at[c, pl.ds(s * RPT, RPT)])


_deg_call = pl.kernel(
    _deg_body,
    out_type=jax.ShapeDtypeStruct((NC, NPAD), jnp.float32),
    mesh=_MESH,
    scratch_types=[
        pltpu.VMEM((NCH, CHUNK), jnp.int32),
        pltpu.VMEM((CHUNK,), jnp.float32),
        pltpu.VMEM((RPT,), jnp.float32),
        pltpu.VMEM_SHARED((NPAD,), jnp.float32),
        pltpu.SemaphoreType.DMA,
    ],
)


def _edge_body(src_hbm, dst_hbm, y_hbm, z_hbm, acc2_hbm,
               sidx_v, didx_v, rows_v, acc_sh, gsem, ssem):
    c = lax.axis_index("c")
    s = lax.axis_index("s")

    @pl.when(c == 0)
    def _():
        pltpu.sync_copy(y_hbm.at[pl.ds(s * RPT, RPT)],
                        acc_sh.at[pl.ds(s * RPT, RPT)])

    @pl.when(c == 1)
    def _():
        pltpu.sync_copy(z_hbm, acc_sh.at[pl.ds(s * RPT, RPT)])

    pltpu.sync_copy(src_hbm.at[c, s, 0], sidx_v.at[0])
    pltpu.sync_copy(dst_hbm.at[c, s, 0], didx_v.at[0])
    plsc.subcore_barrier()

    ngrp = NCH // NBUF

    @pl.loop(0, ngrp)
    def _(p):
        q = lax.rem(p, 2)
        gd = [
            pltpu.async_copy(y_hbm.at[sidx_v.at[q, b]], rows_v.at[b], gsem)
            for b in range(NBUF)
        ]
        for d in gd:
            d.wait()
        sd = [
            pltpu.async_copy(rows_v.at[b], acc_sh.at[didx_v.at[q, b]],
                             ssem, add=True)
            for b in range(NBUF)
        ]

        @pl.when(p + 1 < ngrp)
        def _():
            pltpu.sync_copy(src_hbm.at[c, s, p + 1], sidx_v.at[1 - q])
            pltpu.sync_copy(dst_hbm.at[c, s, p + 1], didx_v.at[1 - q])

        for d in sd:
            d.wait()

    plsc.subcore_barrier()
    pltpu.sync_copy(acc_sh.at[pl.ds(s * RPT, RPT)],
                    acc2_hbm.at[c, pl.ds(s * RPT, RPT)])


_edge_call = pl.kernel(
    _edge_body,
    out_type=jax.ShapeDtypeStruct((NC, NPAD, D), jnp.float32),
    mesh=_MESH,
    scratch_types=[
        pltpu.VMEM((2, NBUF, CHUNK), jnp.int32),
        pltpu.VMEM((2, NBUF, CHUNK), jnp.int32),
        pltpu.VMEM((NBUF, CHUNK, D), jnp.float32),
        pltpu.VMEM_SHARED((NPAD, D), jnp.float32),
        pltpu.SemaphoreType.DMA,
        pltpu.SemaphoreType.DMA,
    ],
)


def _mm_body(x_ref, w_ref, deg2_ref, y_ref):
    xw = jnp.dot(x_ref[...], w_ref[...], preferred_element_type=jnp.float32)
    deg = deg2_ref[0] + deg2_ref[1] + 1.0
    dis = lax.rsqrt(deg)
    y_ref[...] = xw * dis[:, None]


_MM_BLK = 1280


def _mm_call(x_pad, w, deg2):
    grid = NPAD // _MM_BLK
    return pl.pallas_call(
        _mm_body,
        grid=(grid,),
        in_specs=[
            pl.BlockSpec((_MM_BLK, D), lambda i: (i, 0)),
            pl.BlockSpec((D, D), lambda i: (0, 0)),
            pl.BlockSpec((NC, _MM_BLK), lambda i: (0, i)),
        ],
        out_specs=pl.BlockSpec((_MM_BLK, D), lambda i: (i, 0)),
        out_shape=jax.ShapeDtypeStruct((NPAD, D), jnp.float32),
    )(x_pad, w, deg2)


def _post_body(acc2_ref, deg2_ref, b_ref, gamma_ref, beta_ref, out_ref):
    deg = deg2_ref[0, :N] + deg2_ref[1, :N] + 1.0
    dis = lax.rsqrt(deg)[:, None]
    acc = acc2_ref[0, :N, :] + acc2_ref[1, :N, :]
    pre = acc * dis + b_ref[0]
    pre = jnp.where(pre >= 0, pre, 0.01 * pre)
    mean = jnp.mean(pre, axis=0, keepdims=True)
    cent = pre - mean
    var = jnp.mean(cent * cent, axis=0, keepdims=True)
    out_ref[...] = cent * (lax.rsqrt(var + 1e-5) * gamma_ref[0]) + beta_ref[0]


def _post_call(acc2, deg2, b, gamma, beta):
    return pl.pallas_call(
        _post_body,
        out_shape=jax.ShapeDtypeStruct((N, D), jnp.float32),
    )(acc2, deg2, b.reshape(1, D), gamma.reshape(1, D), beta.reshape(1, D))


def kernel(x, edge_index, W, b, gamma, beta):
    ei = edge_index.astype(jnp.int32)
    src, dst = ei[0], ei[1]
    pad_ids = N + (jnp.arange(EPAD - E, dtype=jnp.int32) % (NPAD - N))
    src_p = jnp.concatenate([src, pad_ids])
    dst_p = jnp.concatenate([dst, pad_ids])
    grp = (NC, NS, NCH // NBUF, NBUF, CHUNK)

    x_pad = jnp.pad(x, ((0, NPAD - N), (0, 0)))
    zeros = jnp.zeros((RPT, D), jnp.float32)

    deg2 = _deg_call(dst_p.reshape(NC, NS, NCH, CHUNK))
    y = _mm_call(x_pad, W, deg2)
    acc2 = _edge_call(src_p.reshape(grp), dst_p.reshape(grp), y, zeros)
    return _post_call(acc2, deg2, b, gamma, beta)
